# baseline (speedup 1.0000x reference)
import functools

import jax
import jax.numpy as jnp
from jax import lax
from jax.experimental import pallas as pl
from jax.experimental.pallas import tpu as pltpu
from jax.experimental.pallas import tpu_sc as plsc

EMBED_DIM = 64
CHUNK = 128


@functools.cache
def _gather_fn(B):
    info = plsc.get_sparse_core_info()
    NC, NS = info.num_cores, info.num_subcores
    NW = NC * NS
    b_per_w = B // NW
    k = b_per_w // CHUNK
    mesh = plsc.VectorSubcoreMesh(core_axis_name="c", subcore_axis_name="s")

    @functools.partial(
        pl.kernel,
        mesh=mesh,
        compiler_params=pltpu.CompilerParams(use_tc_tiling_on_sc=False),
        out_type=(
            jax.ShapeDtypeStruct((B, EMBED_DIM), jnp.float32),
            jax.ShapeDtypeStruct((B, EMBED_DIM), jnp.float32),
        ),
        scratch_types=[
            pltpu.VMEM((k, CHUNK), jnp.int32),
            pltpu.VMEM((k, CHUNK), jnp.int32),
            pltpu.VMEM((b_per_w, EMBED_DIM), jnp.float32),
            pltpu.VMEM((b_per_w, EMBED_DIM), jnp.float32),
            pltpu.SemaphoreType.DMA,
            pltpu.SemaphoreType.DMA,
        ],
    )
    def gk(uids, pids, utab, ptab, uout, pout, uidx, pidx, urows, prows,
           usem, psem):
        wid = lax.axis_index("s") * NC + lax.axis_index("c")
        base = wid * b_per_w
        pltpu.sync_copy(uids.at[pl.ds(wid * k, k)], uidx)
        pltpu.sync_copy(pids.at[pl.ds(wid * k, k)], pidx)
        copies = []
        for j in range(k):
            copies.append(pltpu.async_copy(
                utab.at[uidx.at[j]], urows.at[pl.ds(j * CHUNK, CHUNK)], usem))
            copies.append(pltpu.async_copy(
                ptab.at[pidx.at[j]], prows.at[pl.ds(j * CHUNK, CHUNK)], psem))
        for c in copies:
            c.wait()
        pltpu.sync_copy(urows, uout.at[pl.ds(base, b_per_w)])
        pltpu.sync_copy(prows, pout.at[pl.ds(base, b_per_w)])

    return gk


def _mlp_body(u, p, w1u, w1p, b1, w2, b2, w3, b3, o):
    hp = lax.Precision.HIGHEST
    h = (jnp.dot(u[...], w1u[...], precision=hp)
         + jnp.dot(p[...], w1p[...], precision=hp) + b1[...])
    h = jnp.maximum(h, 0.0)
    h = jnp.maximum(jnp.dot(h, w2[...], precision=hp) + b2[...], 0.0)
    o[...] = jax.nn.sigmoid(jnp.dot(h, w3[...], precision=hp) + b3[...])


def _mlp(u_emb, p_emb, W1u, W1p, b1, W2, b2, W3, b3):
    B = u_emb.shape[0]
    BB = 2048
    grid = (B // BB,)
    full = lambda shape: pl.BlockSpec(shape, lambda i: (0, 0))
    return pl.pallas_call(
        _mlp_body,
        grid=grid,
        in_specs=[
            pl.BlockSpec((BB, EMBED_DIM), lambda i: (i, 0)),
            pl.BlockSpec((BB, EMBED_DIM), lambda i: (i, 0)),
            full((EMBED_DIM, 128)),
            full((EMBED_DIM, 128)),
            full((1, 128)),
            full((128, 64)),
            full((1, 64)),
            full((64, 1)),
            full((1, 1)),
        ],
        out_specs=pl.BlockSpec((BB, 1), lambda i: (i, 0)),
        out_shape=jax.ShapeDtypeStruct((B, 1), jnp.float32),
    )(u_emb, p_emb, W1u, W1p, b1, W2, b2, W3, b3)


def kernel(user_ids, post_ids, user_table, post_table, W1, b1, W2, b2, W3, b3):
    B = user_ids.shape[0]
    uids = user_ids.astype(jnp.int32).reshape(B // CHUNK, CHUNK)
    pids = post_ids.astype(jnp.int32).reshape(B // CHUNK, CHUNK)
    u_emb, p_emb = _gather_fn(B)(uids, pids, user_table, post_table)
    return _mlp(u_emb, p_emb, W1[:EMBED_DIM], W1[EMBED_DIM:],
                b1.reshape(1, -1), W2, b2.reshape(1, -1),
                W3, b3.reshape(1, 1))


# speedup vs baseline: 1.5051x; 1.5051x over previous
import functools

import jax
import jax.numpy as jnp
from jax import lax
from jax.experimental import pallas as pl
from jax.experimental.pallas import tpu as pltpu
from jax.experimental.pallas import tpu_sc as plsc

EMBED_DIM = 64


@functools.cache
def _gather_fn(B, V):
    info = plsc.get_sparse_core_info()
    NC, NS = info.num_cores, info.num_subcores
    NW = NC * NS
    b_per_w = B // NW
    mesh = plsc.VectorSubcoreMesh(core_axis_name="c", subcore_axis_name="s")

    @functools.partial(
        pl.kernel,
        mesh=mesh,
        compiler_params=pltpu.CompilerParams(
            use_tc_tiling_on_sc=True, needs_layout_passes=False),
        out_type=(
            jax.ShapeDtypeStruct((B, EMBED_DIM), jnp.float32),
            jax.ShapeDtypeStruct((B, EMBED_DIM), jnp.float32),
        ),
        scratch_types=[
            pltpu.VMEM((b_per_w,), jnp.int32),
            pltpu.VMEM((b_per_w,), jnp.int32),
            pltpu.VMEM((128, EMBED_DIM), jnp.float32),
            pltpu.VMEM((128, EMBED_DIM), jnp.float32),
            pltpu.SemaphoreType.DMA,
            pltpu.SemaphoreType.DMA,
        ],
    )
    def gk(uidx_hbm, pidx_hbm, utab, ptab, uout, pout, uidx_v, pidx_v,
           urows, prows, usem, psem):
        wid = lax.axis_index("s") * NC + lax.axis_index("c")
        base = wid * b_per_w
        pltpu.sync_copy(uidx_hbm.at[pl.ds(base, b_per_w)], uidx_v)
        pltpu.sync_copy(pidx_hbm.at[pl.ds(base, b_per_w)], pidx_v)
        lanes = lax.iota(jnp.int32, 16)

        def blk_body(blk, carry):
            def chunk_body(c, carry2):
                uchunk = uidx_v[pl.ds(blk * 128 + c * 16, 16)]
                pchunk = pidx_v[pl.ds(blk * 128 + c * 16, 16)]
                for j in range(16):
                    us = jnp.max(jnp.where(lanes == j, uchunk, 0))
                    ps = jnp.max(jnp.where(lanes == j, pchunk, 0))
                    ucp = pltpu.async_copy(
                        utab.at[pl.ds(us, 1)],
                        urows.at[pl.ds(c * 16 + j, 1)], usem)
                    pcp = pltpu.async_copy(
                        ptab.at[pl.ds(ps, 1)],
                        prows.at[pl.ds(c * 16 + j, 1)], psem)
                for j in range(16):
                    ucp.wait()
                    pcp.wait()
                return carry2

            lax.fori_loop(0, 8, chunk_body, 0)
            pltpu.sync_copy(urows, uout.at[pl.ds(base + blk * 128, 128)])
            pltpu.sync_copy(prows, pout.at[pl.ds(base + blk * 128, 128)])
            return carry

        lax.fori_loop(0, b_per_w // 128, blk_body, 0)

    return gk


def _mlp_body(u, p, w1u, w1p, b1, w2, b2, w3, b3, o):
    hp = lax.Precision.HIGHEST
    h = (jnp.dot(u[...], w1u[...], precision=hp)
         + jnp.dot(p[...], w1p[...], precision=hp) + b1[...])
    h = jnp.maximum(h, 0.0)
    h = jnp.maximum(jnp.dot(h, w2[...], precision=hp) + b2[...], 0.0)
    o[...] = jax.nn.sigmoid(jnp.dot(h, w3[...], precision=hp) + b3[...])


def _mlp(u_emb, p_emb, W1u, W1p, b1, W2, b2, W3, b3):
    B = u_emb.shape[0]
    BB = 2048
    grid = (B // BB,)
    full = lambda shape: pl.BlockSpec(shape, lambda i: (0, 0))
    return pl.pallas_call(
        _mlp_body,
        grid=grid,
        in_specs=[
            pl.BlockSpec((BB, EMBED_DIM), lambda i: (i, 0)),
            pl.BlockSpec((BB, EMBED_DIM), lambda i: (i, 0)),
            full((EMBED_DIM, 128)),
            full((EMBED_DIM, 128)),
            full((1, 128)),
            full((128, 64)),
            full((1, 64)),
            full((64, 1)),
            full((1, 1)),
        ],
        out_specs=pl.BlockSpec((BB, 1), lambda i: (i, 0)),
        out_shape=jax.ShapeDtypeStruct((B, 1), jnp.float32),
    )(u_emb, p_emb, W1u, W1p, b1, W2, b2, W3, b3)


def kernel(user_ids, post_ids, user_table, post_table, W1, b1, W2, b2, W3, b3):
    B = user_ids.shape[0]
    V = user_table.shape[0]
    uids = user_ids.astype(jnp.int32)
    pids = post_ids.astype(jnp.int32)
    u_emb, p_emb = _gather_fn(B, V)(uids, pids, user_table, post_table)
    return _mlp(u_emb, p_emb, W1[:EMBED_DIM], W1[EMBED_DIM:],
                b1.reshape(1, -1), W2, b2.reshape(1, -1),
                W3, b3.reshape(1, 1))
